# baseline (speedup 1.0000x reference)
import functools

import jax
import jax.numpy as jnp
from jax import lax
from jax.experimental import pallas as pl
from jax.experimental.pallas import tpu as pltpu
from jax.experimental.pallas import tpu_sc as plsc

VS = 100
YP, ZP = 104, 128
MP = VS * YP * ZP
N = 4 * 128 * 128
HB = MP // 2
DUMP = 3072
ACC_N = HB + DUMP
NTILES = 16
TSLICE = ACC_N // NTILES
ZCH = TSLICE // 4
PPT = N // NTILES
NROW = PPT // 128
XB = 4


def _vox_idx_body(p_ref, m_ref, r_ref, o_ref):
    p = p_ref[...]
    m = m_ref[...].reshape(1, 3)[:, :, None]
    r = r_ref[...].reshape(1, 3)[:, :, None]
    e = jnp.clip(jnp.floor((p - m) / r), 0.0, float(VS - 1)).astype(jnp.int32)
    o_ref[...] = (e[:, 0:1, :] * (YP * ZP) + e[:, 1:2, :] * ZP + e[:, 2:3, :])


def _vox_idx(pcd_r, bb_min, res):
    B = pcd_r.shape[0]
    return pl.pallas_call(
        _vox_idx_body,
        grid=(B,),
        in_specs=[
            pl.BlockSpec((1, 3, N), lambda b: (b, 0, 0)),
            pl.BlockSpec((1, 1, 3), lambda b: (b, 0, 0)),
            pl.BlockSpec((1, 1, 3), lambda b: (b, 0, 0)),
        ],
        out_specs=pl.BlockSpec((1, 1, N), lambda b: (b, 0, 0)),
        out_shape=jax.ShapeDtypeStruct((B, 1, N), jnp.int32),
    )(pcd_r, bb_min.reshape(B, 1, 3), res.reshape(B, 1, 3))


def _fin_compute(s, j, o_ref):
    cnt = s[:, 6, :, :VS, :VS]
    rden = 1.0 / jnp.maximum(cnt, 1.0)
    for c in range(6):
        o_ref[:, c] = s[:, c, :, :VS, :VS] * rden
    dv = jnp.float32(1.0 / (VS - 1))
    sh = (1, XB, VS, VS)
    xg = (j * XB + lax.broadcasted_iota(jnp.int32, sh, 1)).astype(jnp.float32)
    o_ref[:, 6] = xg * dv
    o_ref[:, 7] = lax.broadcasted_iota(jnp.int32, sh, 2).astype(jnp.float32) * dv
    o_ref[:, 8] = lax.broadcasted_iota(jnp.int32, sh, 3).astype(jnp.float32) * dv
    o_ref[:, 9] = (cnt > 0.0).astype(jnp.float32)


def _finalize4_body(s0, s1, s2, s3, o_ref):
    b = pl.program_id(0)
    j = pl.program_id(1)
    for k, sk in enumerate((s0, s1, s2, s3)):
        @pl.when(b == k)
        def _(sk=sk):
            _fin_compute(sk[...], j, o_ref)


def _finalize4(sums_list):
    def in_spec(k):
        return pl.BlockSpec(
            (1, 7, XB, YP, ZP),
            lambda b, j, k=k: (0, 0, jnp.where(b == k, j, 0), 0, 0))
    return pl.pallas_call(
        _finalize4_body,
        grid=(4, VS // XB),
        in_specs=[in_spec(k) for k in range(4)],
        out_specs=pl.BlockSpec((1, 10, XB, VS, VS), lambda b, j: (b, 0, j, 0, 0)),
        out_shape=jax.ShapeDtypeStruct((4, 10, VS, VS, VS), jnp.float32),
    )(*sums_list)


def _sc_scatter(idx1, pcd1, rgb1, bq):
    mesh = plsc.VectorSubcoreMesh(core_axis_name="c", subcore_axis_name="s")

    @functools.partial(
        pl.kernel,
        out_type=jax.ShapeDtypeStruct((7 * MP,), jnp.float32),
        mesh=mesh,
        scratch_types=[
            pltpu.VMEM_SHARED((ACC_N,), jnp.float32),
            pltpu.VMEM_SHARED((ACC_N,), jnp.float32),
            pltpu.VMEM((PPT,), jnp.int32),
            pltpu.VMEM((PPT,), jnp.int32),
            pltpu.VMEM((PPT,), jnp.float32),
            pltpu.VMEM((ZCH,), jnp.float32),
            pltpu.VMEM((PPT,), jnp.float32),
            pltpu.SemaphoreType.DMA,
        ],
    )
    def sck(idx_hbm, pcd_hbm, rgb_hbm, out_hbm,
            a0, a1, idx_s, lidx, vals, zbuf, ones, zsem):
        cid = lax.axis_index("c")
        sid = lax.axis_index("s")
        hbase = cid * HB
        pbase = sid * PPT
        accs = (a0, a1)

        zv = jnp.zeros((16,), jnp.float32)

        def _zfill(i, _):
            zbuf[pl.ds(i * 16, 16)] = zv
            return 0
        lax.fori_loop(0, ZCH // 16, _zfill, 0)
        ov = jnp.full((16,), 1.0, jnp.float32)

        def _ofill(g, _):
            ones[pl.ds(g * 16, 16)] = ov
            return 0
        lax.fori_loop(0, PPT // 16, _ofill, 0)

        i16 = lax.iota(jnp.int32, 16)

        def localize(b):
            pltpu.sync_copy(idx_hbm.at[pl.ds(b * N + pbase, PPT)], idx_s)

            def _l(g, _):
                iv = idx_s[pl.ds(g * 16, 16)]
                li = iv - hbase
                inr = (li >= 0) & (li < HB)
                dump = HB + ((g * 16 + i16) & (2048 - 1))
                lidx[pl.ds(g * 16, 16)] = jnp.where(inr, li, dump)
                return 0
            lax.fori_loop(0, PPT // 16, _l, 0)

        def run_round(b, chans):
            plsc.subcore_barrier()
            handles = [
                pltpu.async_copy(
                    zbuf, a.at[pl.ds(sid * TSLICE + i * ZCH, ZCH)], zsem)
                for a in accs[:len(chans)]
                for i in range(TSLICE // ZCH)
            ]
            for h in handles:
                h.wait()
            plsc.subcore_barrier()
            for k, ch in enumerate(chans):
                if ch == 6:
                    pltpu.sync_copy(ones, accs[k].at[lidx], add=True)
                else:
                    h = pcd_hbm if ch < 3 else rgb_hbm
                    pltpu.sync_copy(
                        h.at[pl.ds((b * 3 + ch % 3) * N + pbase, PPT)], vals)
                    pltpu.sync_copy(vals, accs[k].at[lidx], add=True)
            plsc.subcore_barrier()
            for k, ch in enumerate(chans):
                @pl.when(sid == k)
                def _():
                    pltpu.sync_copy(
                        accs[k].at[pl.ds(0, HB)],
                        out_hbm.at[pl.ds(ch * MP + cid * HB, HB)])

        localize(bq)
        run_round(bq, (6, 0))
        run_round(bq, (1, 2))
        run_round(bq, (3, 4))
        run_round(bq, (5,))
        plsc.subcore_barrier()

    return sck(idx1, pcd1, rgb1)


def kernel(rgb, pcd, bounds, depth, proprio, camera_extrinsics,
           camera_intrinsics, lang_goal_emb, lang_token_embs):
    B = pcd.shape[1]
    pcd_r = jnp.transpose(pcd, (1, 2, 0, 3, 4)).reshape(B, 3, N)
    rgb_r = jnp.transpose(rgb, (1, 2, 0, 3, 4)).reshape(B, 3, N)
    bb_min = bounds[:, :3]
    res = (bounds[:, 3:] - bb_min) / float(VS)
    idxv = _vox_idx(pcd_r, bb_min, res)
    idx1 = idxv.reshape(B * N)
    pcd1 = pcd_r.reshape(B * 3 * N)
    rgb1 = rgb_r.reshape(B * 3 * N)
    sums_list = [
        _sc_scatter(idx1, pcd1, rgb1, b).reshape(1, 7, VS, YP, ZP)
        for b in range(B)
    ]
    return _finalize4(sums_list)


# speedup vs baseline: 1.1557x; 1.1557x over previous
import functools

import jax
import jax.numpy as jnp
from jax import lax
from jax.experimental import pallas as pl
from jax.experimental.pallas import tpu as pltpu
from jax.experimental.pallas import tpu_sc as plsc

VS = 100
YP, ZP = 104, 128
MP = VS * YP * ZP
N = 4 * 128 * 128
HB = MP // 2
DUMP = 3072
ACC_N = HB + DUMP
NTILES = 16
TSLICE = ACC_N // NTILES
ZCH = TSLICE // 4
PPT = N // NTILES
NROW = PPT // 128
XB = 4


def _vox_idx_body(p_ref, m_ref, r_ref, o_ref):
    p = p_ref[...]
    m = m_ref[...].reshape(1, 3)[:, :, None]
    r = r_ref[...].reshape(1, 3)[:, :, None]
    e = jnp.clip(jnp.floor((p - m) / r), 0.0, float(VS - 1)).astype(jnp.int32)
    o_ref[...] = (e[:, 0:1, :] * (YP * ZP) + e[:, 1:2, :] * ZP + e[:, 2:3, :])


def _vox_idx(pcd_r, bb_min, res):
    B = pcd_r.shape[0]
    return pl.pallas_call(
        _vox_idx_body,
        grid=(B,),
        in_specs=[
            pl.BlockSpec((1, 3, N), lambda b: (b, 0, 0)),
            pl.BlockSpec((1, 1, 3), lambda b: (b, 0, 0)),
            pl.BlockSpec((1, 1, 3), lambda b: (b, 0, 0)),
        ],
        out_specs=pl.BlockSpec((1, 1, N), lambda b: (b, 0, 0)),
        out_shape=jax.ShapeDtypeStruct((B, 1, N), jnp.int32),
    )(pcd_r, bb_min.reshape(B, 1, 3), res.reshape(B, 1, 3))


def _fin_compute(s, j, o_ref):
    cnt = s[:, 6, :, :VS, :VS]
    rden = 1.0 / jnp.maximum(cnt, 1.0)
    for c in range(6):
        o_ref[:, c] = s[:, c, :, :VS, :VS] * rden
    dv = jnp.float32(1.0 / (VS - 1))
    sh = (1, XB, VS, VS)
    xg = (j * XB + lax.broadcasted_iota(jnp.int32, sh, 1)).astype(jnp.float32)
    o_ref[:, 6] = xg * dv
    o_ref[:, 7] = lax.broadcasted_iota(jnp.int32, sh, 2).astype(jnp.float32) * dv
    o_ref[:, 8] = lax.broadcasted_iota(jnp.int32, sh, 3).astype(jnp.float32) * dv
    o_ref[:, 9] = (cnt > 0.0).astype(jnp.float32)


def _fin_body(s_ref, o_ref):
    _fin_compute(s_ref[...], pl.program_id(0), o_ref)


def _fin_body_carry(s_ref, c_ref, o_ref):
    del c_ref
    _fin_compute(s_ref[...], pl.program_id(0), o_ref)


def _finalize_b(sums_b, carry, b, B):
    out_shape = jax.ShapeDtypeStruct((B, 10, VS, VS, VS), jnp.float32)
    in_specs = [pl.BlockSpec((1, 7, XB, YP, ZP), lambda j: (0, 0, j, 0, 0))]
    out_spec = pl.BlockSpec((1, 10, XB, VS, VS), lambda j, b=b: (b, 0, j, 0, 0))
    if carry is None:
        return pl.pallas_call(
            _fin_body, grid=(VS // XB,), in_specs=in_specs,
            out_specs=out_spec, out_shape=out_shape)(sums_b)
    in_specs.append(pl.BlockSpec(memory_space=pl.ANY))
    return pl.pallas_call(
        _fin_body_carry, grid=(VS // XB,), in_specs=in_specs,
        out_specs=out_spec, out_shape=out_shape,
        input_output_aliases={1: 0})(sums_b, carry)


def _sc_scatter(idx1, pcd1, rgb1, bq):
    mesh = plsc.VectorSubcoreMesh(core_axis_name="c", subcore_axis_name="s")

    @functools.partial(
        pl.kernel,
        out_type=jax.ShapeDtypeStruct((7 * MP,), jnp.float32),
        mesh=mesh,
        scratch_types=[
            pltpu.VMEM_SHARED((ACC_N,), jnp.float32),
            pltpu.VMEM_SHARED((ACC_N,), jnp.float32),
            pltpu.VMEM((PPT,), jnp.int32),
            pltpu.VMEM((PPT,), jnp.int32),
            pltpu.VMEM((PPT,), jnp.float32),
            pltpu.VMEM((ZCH,), jnp.float32),
            pltpu.VMEM((PPT,), jnp.float32),
            pltpu.SemaphoreType.DMA,
        ],
    )
    def sck(idx_hbm, pcd_hbm, rgb_hbm, out_hbm,
            a0, a1, idx_s, lidx, vals, zbuf, ones, zsem):
        cid = lax.axis_index("c")
        sid = lax.axis_index("s")
        hbase = cid * HB
        pbase = sid * PPT
        accs = (a0, a1)

        zv = jnp.zeros((16,), jnp.float32)

        def _zfill(i, _):
            zbuf[pl.ds(i * 16, 16)] = zv
            return 0
        lax.fori_loop(0, ZCH // 16, _zfill, 0)
        ov = jnp.full((16,), 1.0, jnp.float32)

        def _ofill(g, _):
            ones[pl.ds(g * 16, 16)] = ov
            return 0
        lax.fori_loop(0, PPT // 16, _ofill, 0)

        i16 = lax.iota(jnp.int32, 16)

        def localize(b):
            pltpu.sync_copy(idx_hbm.at[pl.ds(b * N + pbase, PPT)], idx_s)

            def _l(g, _):
                iv = idx_s[pl.ds(g * 16, 16)]
                li = iv - hbase
                inr = (li >= 0) & (li < HB)
                dump = HB + ((g * 16 + i16) & (2048 - 1))
                lidx[pl.ds(g * 16, 16)] = jnp.where(inr, li, dump)
                return 0
            lax.fori_loop(0, PPT // 16, _l, 0)

        def run_round(b, chans):
            plsc.subcore_barrier()
            handles = [
                pltpu.async_copy(
                    zbuf, a.at[pl.ds(sid * TSLICE + i * ZCH, ZCH)], zsem)
                for a in accs[:len(chans)]
                for i in range(TSLICE // ZCH)
            ]
            for h in handles:
                h.wait()
            plsc.subcore_barrier()
            for k, ch in enumerate(chans):
                if ch == 6:
                    pltpu.sync_copy(ones, accs[k].at[lidx], add=True)
                else:
                    h = pcd_hbm if ch < 3 else rgb_hbm
                    pltpu.sync_copy(
                        h.at[pl.ds((b * 3 + ch % 3) * N + pbase, PPT)], vals)
                    pltpu.sync_copy(vals, accs[k].at[lidx], add=True)
            plsc.subcore_barrier()
            for k, ch in enumerate(chans):
                @pl.when(sid == k)
                def _():
                    pltpu.sync_copy(
                        accs[k].at[pl.ds(0, HB)],
                        out_hbm.at[pl.ds(ch * MP + cid * HB, HB)])

        localize(bq)
        run_round(bq, (6, 0))
        run_round(bq, (1, 2))
        run_round(bq, (3, 4))
        run_round(bq, (5,))
        plsc.subcore_barrier()

    return sck(idx1, pcd1, rgb1)


def kernel(rgb, pcd, bounds, depth, proprio, camera_extrinsics,
           camera_intrinsics, lang_goal_emb, lang_token_embs):
    B = pcd.shape[1]
    pcd_r = jnp.transpose(pcd, (1, 2, 0, 3, 4)).reshape(B, 3, N)
    rgb_r = jnp.transpose(rgb, (1, 2, 0, 3, 4)).reshape(B, 3, N)
    bb_min = bounds[:, :3]
    res = (bounds[:, 3:] - bb_min) / float(VS)
    idxv = _vox_idx(pcd_r, bb_min, res)
    idx1 = idxv.reshape(B * N)
    pcd1 = pcd_r.reshape(B * 3 * N)
    rgb1 = rgb_r.reshape(B * 3 * N)
    out = None
    for b in range(B):
        sums_b = _sc_scatter(idx1, pcd1, rgb1, b).reshape(1, 7, VS, YP, ZP)
        out = _finalize_b(sums_b, out, b, B)
    return out
